# baseline (speedup 1.0000x reference)
import functools

import jax
import jax.numpy as jnp
from jax import lax
from jax.experimental import pallas as pl
from jax.experimental.pallas import tpu as pltpu
from jax.experimental.pallas import tpu_sc as plsc

N = 500000
D = 128
NT = 218
NL = 100
SPLIT = 150
NW = 32
TPW = 7
NG = NW * TPW
P = NG * NL
LROWS = TPW * NL
CH = 800
GB = 64
IB = CH + GB


def _scalar(v):
    return v[0] if getattr(v, "ndim", 0) == 1 else v


def _keys_body(t_ref, l_ref, k_ref):
    t = t_ref[...]
    l = l_ref[...]
    k_ref[...] = ((t & 31) * TPW + (t >> 5)) * NL + l


_keys_call = pl.pallas_call(
    _keys_body,
    out_shape=jax.ShapeDtypeStruct((500, 1000), jnp.int32),
)


NC = N // CH


def _hist_body(x_hbm, keys_hbm, s_out, c_out, q_out,
               s_loc, c_loc, q_loc, kbuf, ibuf, lbuf, grows,
               ksem0, ksem1, gsem0, gsem1):
    wid = lax.axis_index("s") * 2 + lax.axis_index("c")
    lo = wid * LROWS
    zf = jnp.zeros((16,), jnp.float32)
    iot = lax.broadcasted_iota(jnp.int32, (16,), 0)
    cone = jnp.where(iot == 0, jnp.ones((16,), jnp.float32), zf)
    ksems = (ksem0, ksem1)
    gsems = (gsem0, gsem1)

    def zs(i, _):
        for c in range(8):
            s_loc[pl.ds(i * 128 + c * 16, 16)] = zf
        return 0

    lax.fori_loop(0, 704, zs, 0)

    def zq(i, _):
        q_loc[pl.ds(i * 16, 16)] = zf
        return 0

    lax.fori_loop(0, 11280 // 16, zq, 0)

    def zc(i, _):
        c_loc[pl.ds(i * 16, 16)] = zf
        return 0

    lax.fori_loop(0, 5648 // 16, zc, 0)

    def issue_key(ci, slot):
        pltpu.async_copy(keys_hbm.at[pl.ds(ci * CH, CH)],
                         kbuf.at[pl.ds(slot * CH, CH)], ksems[slot])

    def wait_key(slot):
        pltpu.make_async_copy(keys_hbm.at[pl.ds(0, CH)],
                              kbuf.at[pl.ds(slot * CH, CH)],
                              ksems[slot]).wait()

    def scan(ci, slot):
        base = ci * CH

        def sb(o, cursor):
            for q in range(5):
                off = (o * 5 + q) * 16
                kv = kbuf[pl.ds(slot * CH + off, 16)]
                lov = jnp.full((16,), lo, jnp.int32)
                m = (kv >= lov) & (kv < lov + LROWS)
                rid = jnp.full((16,), base + off, jnp.int32) + iot
                plsc.store_compressed(ibuf.at[pl.ds(slot * IB + cursor, 16)], rid,
                                      mask=m)
                plsc.store_compressed(lbuf.at[pl.ds(slot * IB + cursor, 16)],
                                      kv - lov, mask=m)
                cursor = cursor + _scalar(plsc.all_reduce_population_count(m))
            return cursor

        cnt = lax.fori_loop(0, (CH // 16) // 5, sb, jnp.int32(0))
        for k in range(GB // 16):
            ibuf[pl.ds(slot * IB + cnt + k * 16, 16)] = jnp.zeros((16,),
                                                                  jnp.int32)
            lbuf[pl.ds(slot * IB + cnt + k * 16, 16)] = jnp.full((16,), LROWS,
                                                                 jnp.int32)
        return cnt

    def issue_g0(slot):
        pltpu.async_copy(x_hbm.at[ibuf.at[pl.ds(slot * IB, GB)]],
                         grows.at[slot], gsems[slot])

    def wait_g0(slot):
        pltpu.make_async_copy(x_hbm.at[ibuf.at[pl.ds(slot * IB, GB)]],
                              grows.at[slot], gsems[slot]).wait()

    def accum16(slot, lboff, u):
        lrowv = lbuf[pl.ds(slot * IB + lboff + u * 16, 16)]
        for r in range(16):
            lb = lrowv[r]
            row = u * 16 + r
            sq = zf
            for c in range(8):
                xc = grows[slot, row, pl.ds(c * 16, 16)]
                ssl = pl.ds(lb * 128 + c * 16, 16)
                s_loc[ssl] = s_loc[ssl] + xc
                sq = sq + xc * xc
            qsl = pl.ds(lb * 16, 16)
            q_loc[qsl] = q_loc[qsl] + sq
            csl = pl.ds(lb * 8, 16)
            c_loc[csl] = c_loc[csl] + cone
        return 0

    def extra_batches(slot, cnt):
        nb = (cnt + GB - 1) // GB

        def eb(b, _):
            pltpu.async_copy(x_hbm.at[ibuf.at[pl.ds(slot * IB + b * GB, GB)]],
                             grows.at[slot], gsems[slot]).wait()
            lax.fori_loop(0, GB // 16,
                          lambda u, _: accum16(slot, b * GB, u), 0)
            return 0

        lax.fori_loop(1, nb, eb, 0)

    def process(ci, slot):
        wait_key(slot)
        cnt = scan(ci, slot)
        issue_g0(slot)

        @pl.when(ci + 2 < NC)
        def _():
            issue_key(ci + 2, slot)

        return cnt

    def drain(slot, cnt):
        wait_g0(slot)
        lax.fori_loop(0, GB // 16, lambda u, _: accum16(slot, 0, u), 0)
        extra_batches(slot, cnt)

    issue_key(0, 0)
    issue_key(1, 1)

    def pair(j, _):
        a = 2 * j
        cnt_a = process(a, 0)
        cnt_b = process(a + 1, 1)
        drain(0, cnt_a)
        drain(1, cnt_b)
        return 0

    lax.fori_loop(0, NC // 2, pair, 0)
    if NC % 2:
        wait_key(0)
        cnt = scan(NC - 1, 0)
        issue_g0(0)
        drain(0, cnt)

    pltpu.sync_copy(s_loc.at[pl.ds(0, LROWS * 128)],
                    s_out.at[pl.ds(lo * 128, LROWS * 128)])
    pltpu.sync_copy(c_loc.at[pl.ds(0, LROWS * 8)],
                    c_out.at[pl.ds(wid * LROWS * 8, LROWS * 8)])
    pltpu.sync_copy(q_loc.at[pl.ds(0, LROWS * 16)],
                    q_out.at[pl.ds(wid * LROWS * 16, LROWS * 16)])


@functools.lru_cache(maxsize=None)
def _get_hist_call():
  return functools.partial(
    pl.kernel,
    mesh=plsc.VectorSubcoreMesh(core_axis_name="c", subcore_axis_name="s"),
    compiler_params=pltpu.CompilerParams(needs_layout_passes=False),
    out_type=[
        jax.ShapeDtypeStruct((P * D,), jnp.float32),
        jax.ShapeDtypeStruct((P * 8,), jnp.float32),
        jax.ShapeDtypeStruct((P * 16,), jnp.float32),
    ],
    scratch_types=[
        pltpu.VMEM((704 * D,), jnp.float32),
        pltpu.VMEM((5648,), jnp.float32),
        pltpu.VMEM((11280,), jnp.float32),
        pltpu.VMEM((2 * CH,), jnp.int32),
        pltpu.VMEM((2 * (CH + GB),), jnp.int32),
        pltpu.VMEM((2 * (CH + GB),), jnp.int32),
        pltpu.VMEM((2, GB, D), jnp.float32),
        pltpu.SemaphoreType.DMA,
        pltpu.SemaphoreType.DMA,
        pltpu.SemaphoreType.DMA,
        pltpu.SemaphoreType.DMA,
    ],
  )(_hist_body)


def _stats_body(s_ref, wc_ref, wq_ref, mean_ref, tsum_ref, tsc_ref):
    S = s_ref[0]
    C = wc_ref[0][:, 0:1]
    Q = jnp.sum(wq_ref[0], axis=1, keepdims=True)
    mean = S / jnp.maximum(C, 1.0)
    mean_ref[...] = mean[None]
    tsum = jnp.sum(S, axis=0, keepdims=True)
    tsum_ref[...] = tsum[None]
    tcnt = jnp.sum(C)
    tQ = jnp.sum(Q)
    tmean = tsum / jnp.maximum(tcnt, 1.0)
    d2 = jnp.sum((mean - tmean) ** 2, axis=1, keepdims=True)
    msqn = jnp.sum(C * d2)
    m2 = jnp.sum(mean * mean, axis=1, keepdims=True)
    rsqn = tQ - jnp.sum(C * m2)
    io = lax.broadcasted_iota(jnp.int32, (1, 1, 8), 2)
    tsc_ref[...] = jnp.where(
        io == 0, tcnt,
        jnp.where(io == 1, tQ, jnp.where(io == 2, msqn, rsqn)))


_stats_call = pl.pallas_call(
    _stats_body,
    grid=(NG,),
    in_specs=[
        pl.BlockSpec((1, NL, D), lambda g: (g, 0, 0)),
        pl.BlockSpec((1, NL, 8), lambda g: (g, 0, 0)),
        pl.BlockSpec((1, NL, 16), lambda g: (g, 0, 0)),
    ],
    out_specs=[
        pl.BlockSpec((1, NL, D), lambda g: (g, 0, 0)),
        pl.BlockSpec((1, 1, D), lambda g: (g, 0, 0)),
        pl.BlockSpec((1, 1, 8), lambda g: (g, 0, 0)),
    ],
    out_shape=[
        jax.ShapeDtypeStruct((NG, NL, D), jnp.float32),
        jax.ShapeDtypeStruct((NG, 1, D), jnp.float32),
        jax.ShapeDtypeStruct((NG, 1, 8), jnp.float32),
    ],
)


def _alpha_body(tsum_ref, tsc_ref, ab_ref):
    tcnt = tsc_ref[:, 0:1]
    tQ = tsc_ref[:, 1:2]
    msqn = tsc_ref[:, 2:3]
    rsqn = tsc_ref[:, 3:4]
    gi = lax.broadcasted_iota(jnp.int32, (NG, 1), 0)
    tg = (gi // TPW) + 32 * (gi % TPW)
    is_tr = tg < SPLIT
    is_te = (tg >= SPLIT) & (tg < NT)
    te_cnt = jnp.sum(jnp.where(is_te, tcnt, 0.0))
    te_sum = jnp.sum(jnp.where(is_te, tsum_ref[...], 0.0), axis=0,
                     keepdims=True)
    te_Q = jnp.sum(jnp.where(is_te, tQ, 0.0))
    te_mean = te_sum / jnp.maximum(te_cnt, 1.0)
    test_var = (te_Q - te_cnt * jnp.sum(te_mean * te_mean)) / jnp.maximum(
        1.0, te_cnt - 1.0)
    den = jnp.maximum(1.0, tcnt - 1.0)
    msq = msqn / den
    rsq = rsqn / den
    a_sq = (test_var - msq) / jnp.maximum(1e-6, rsq)
    pos = a_sq > 0
    alpha = jnp.where(pos, jnp.sqrt(jnp.where(pos, a_sq, 1.0)), 0.0)
    A = jnp.where(is_tr, alpha, 1.0)
    B = jnp.where(is_tr, 1.0 - alpha, 0.0)
    io = lax.broadcasted_iota(jnp.int32, (NG, 8), 1)
    ab_ref[...] = jnp.where(io == 0, A, jnp.where(io == 1, B, 0.0))


_alpha_call = pl.pallas_call(
    _alpha_body,
    out_shape=jax.ShapeDtypeStruct((NG, 8), jnp.float32),
)


def _norm_body(x_hbm, keys_hbm, mean_hbm, a_hbm, b_hbm, out_hbm,
               av, bv, kb, mrows, xrows, orows, sem):
    wid = lax.axis_index("s") * 2 + lax.axis_index("c")
    pltpu.sync_copy(a_hbm, av)
    pltpu.sync_copy(b_hbm, bv)
    per = N // NW
    start = (wid * per // GB) * GB
    nxt = jnp.where(wid == NW - 1, N, ((wid + 1) * per // GB) * GB)
    nb = (nxt - start + GB - 1) // GB

    def cb(b, _):
        base = jnp.minimum(start + b * GB, N - GB)
        pltpu.sync_copy(keys_hbm.at[pl.ds(base, GB)], kb)
        cp = pltpu.async_copy(mean_hbm.at[kb], mrows, sem)
        pltpu.sync_copy(x_hbm.at[pl.ds(base, GB)], xrows)
        cp.wait()

        def ub(u, _):
            kv = kb[pl.ds(u * 16, 16)]
            tix = kv // NL
            aa = plsc.load_gather(av, [tix])
            bb = plsc.load_gather(bv, [tix])
            for r in range(16):
                af = jnp.full((16,), aa[r], jnp.float32)
                bf = jnp.full((16,), bb[r], jnp.float32)
                row = u * 16 + r
                for c in range(8):
                    sl = pl.ds(c * 16, 16)
                    orows[row, sl] = bf * mrows[row, sl] + af * xrows[row, sl]
            return 0

        lax.fori_loop(0, GB // 16, ub, 0)
        pltpu.sync_copy(orows, out_hbm.at[pl.ds(base, GB)])
        return 0

    lax.fori_loop(0, nb, cb, 0)


@functools.lru_cache(maxsize=None)
def _get_norm_call():
  return functools.partial(
    pl.kernel,
    mesh=plsc.VectorSubcoreMesh(core_axis_name="c", subcore_axis_name="s"),
    compiler_params=pltpu.CompilerParams(needs_layout_passes=False),
    out_type=jax.ShapeDtypeStruct((N, D), jnp.float32),
    scratch_types=[
        pltpu.VMEM((NG,), jnp.float32),
        pltpu.VMEM((NG,), jnp.float32),
        pltpu.VMEM((GB,), jnp.int32),
        pltpu.VMEM((GB, D), jnp.float32),
        pltpu.VMEM((GB, D), jnp.float32),
        pltpu.VMEM((GB, D), jnp.float32),
        pltpu.SemaphoreType.DMA,
    ],
  )(_norm_body)


def kernel(x, labels, times):
    t32 = times.astype(jnp.int32)
    l32 = labels.astype(jnp.int32)
    keys = _keys_call(t32.reshape(500, 1000),
                      l32.reshape(500, 1000)).reshape(N)
    Sf, Cf, Qf = _get_hist_call()(x, keys)
    S3 = Sf.reshape(NG, NL, D)
    C3 = Cf.reshape(NG, NL, 8)
    Q3 = Qf.reshape(NG, NL, 16)
    mean3, tsum3, tsc3 = _stats_call(S3, C3, Q3)
    ab = _alpha_call(tsum3.reshape(NG, D), tsc3.reshape(NG, 8))
    A = ab[:, 0]
    B = ab[:, 1]
    return _get_norm_call()(x, keys, mean3.reshape(P, D), A, B)


# speedup vs baseline: 5.1328x; 5.1328x over previous
import functools

import jax
import jax.numpy as jnp
from jax import lax
from jax.experimental import pallas as pl
from jax.experimental.pallas import tpu as pltpu
from jax.experimental.pallas import tpu_sc as plsc

N = 500000
D = 128
NT = 218
NL = 100
SPLIT = 150
NW = 32
TPW = 7
NG = NW * TPW
P = NG * NL
LROWS = TPW * NL
CH = 2000
GB = 32
IB = CH + GB


def _scalar(v):
    return v[0] if getattr(v, "ndim", 0) == 1 else v


def _keys_body(t_ref, l_ref, k_ref):
    t = t_ref[...]
    l = l_ref[...]
    k_ref[...] = ((t & 31) * TPW + (t >> 5)) * NL + l


_keys_call = pl.pallas_call(
    _keys_body,
    out_shape=jax.ShapeDtypeStruct((500, 1000), jnp.int32),
)


NC = N // CH


def _hist_body(x_hbm, keys_hbm, s_out, c_out, q_out,
               s_loc, c_loc, q_loc, kbuf, ibuf, lbuf, grows,
               ksem0, ksem1, gsem0, gsem1):
    wid = lax.axis_index("s") * 2 + lax.axis_index("c")
    lo = wid * LROWS
    zf = jnp.zeros((16,), jnp.float32)
    iot = lax.broadcasted_iota(jnp.int32, (16,), 0)
    cone = jnp.where(iot == 0, jnp.ones((16,), jnp.float32), zf)
    ksems = (ksem0, ksem1)
    gsems = (gsem0, gsem1)

    def zs(i, _):
        for c in range(8):
            s_loc[pl.ds(i * 128 + c * 16, 16)] = zf
        return 0

    lax.fori_loop(0, 704, zs, 0)

    def zq(i, _):
        q_loc[pl.ds(i * 16, 16)] = zf
        return 0

    lax.fori_loop(0, 11280 // 16, zq, 0)

    def zc(i, _):
        c_loc[pl.ds(i * 16, 16)] = zf
        return 0

    lax.fori_loop(0, 5648 // 16, zc, 0)

    def issue_key(ci, slot):
        pltpu.async_copy(keys_hbm.at[pl.ds(ci * CH, CH)],
                         kbuf.at[pl.ds(slot * CH, CH)], ksems[slot])

    def wait_key(slot):
        pltpu.make_async_copy(keys_hbm.at[pl.ds(0, CH)],
                              kbuf.at[pl.ds(slot * CH, CH)],
                              ksems[slot]).wait()

    def scan(ci, slot):
        base = ci * CH

        def sb(o, cursor):
            for q in range(5):
                off = (o * 5 + q) * 16
                kv = kbuf[pl.ds(slot * CH + off, 16)]
                lov = jnp.full((16,), lo, jnp.int32)
                m = (kv >= lov) & (kv < lov + LROWS)
                rid = jnp.full((16,), base + off, jnp.int32) + iot
                plsc.store_compressed(ibuf.at[pl.ds(slot * IB + cursor, 16)], rid,
                                      mask=m)
                plsc.store_compressed(lbuf.at[pl.ds(slot * IB + cursor, 16)],
                                      kv - lov, mask=m)
                cursor = cursor + _scalar(plsc.all_reduce_population_count(m))
            return cursor

        cnt = lax.fori_loop(0, (CH // 16) // 5, sb, jnp.int32(0))
        for k in range(GB // 16):
            ibuf[pl.ds(slot * IB + cnt + k * 16, 16)] = jnp.zeros((16,),
                                                                  jnp.int32)
            lbuf[pl.ds(slot * IB + cnt + k * 16, 16)] = jnp.full((16,), LROWS,
                                                                 jnp.int32)
        return cnt

    def issue_g0(slot):
        pltpu.async_copy(x_hbm.at[ibuf.at[pl.ds(slot * IB, GB)]],
                         grows.at[slot], gsems[slot])

    def wait_g0(slot):
        pltpu.make_async_copy(x_hbm.at[ibuf.at[pl.ds(slot * IB, GB)]],
                              grows.at[slot], gsems[slot]).wait()

    def accum16(slot, lboff, u):
        lrowv = lbuf[pl.ds(slot * IB + lboff + u * 16, 16)]
        for r in range(16):
            lb = lrowv[r]
            row = u * 16 + r
            sq = zf
            for c in range(8):
                xc = grows[slot, row, pl.ds(c * 16, 16)]
                ssl = pl.ds(lb * 128 + c * 16, 16)
                s_loc[ssl] = s_loc[ssl] + xc
                sq = sq + xc * xc
            qsl = pl.ds(lb * 16, 16)
            q_loc[qsl] = q_loc[qsl] + sq
            csl = pl.ds(lb * 8, 16)
            c_loc[csl] = c_loc[csl] + cone
        return 0

    def extra_batches(slot, cnt):
        nb = (cnt + GB - 1) // GB

        def eb(b, _):
            pltpu.async_copy(x_hbm.at[ibuf.at[pl.ds(slot * IB + b * GB, GB)]],
                             grows.at[slot], gsems[slot]).wait()
            lax.fori_loop(0, GB // 16,
                          lambda u, _: accum16(slot, b * GB, u), 0)
            return 0

        lax.fori_loop(1, nb, eb, 0)

    def process(ci, slot):
        wait_key(slot)
        cnt = scan(ci, slot)
        issue_g0(slot)

        @pl.when(ci + 2 < NC)
        def _():
            issue_key(ci + 2, slot)

        return cnt

    def drain(slot, cnt):
        wait_g0(slot)
        lax.fori_loop(0, GB // 16, lambda u, _: accum16(slot, 0, u), 0)
        extra_batches(slot, cnt)

    issue_key(0, 0)
    issue_key(1, 1)

    def pair(j, _):
        a = 2 * j
        cnt_a = process(a, 0)
        cnt_b = process(a + 1, 1)
        drain(0, cnt_a)
        drain(1, cnt_b)
        return 0

    lax.fori_loop(0, NC // 2, pair, 0)
    if NC % 2:
        wait_key(0)
        cnt = scan(NC - 1, 0)
        issue_g0(0)
        drain(0, cnt)

    pltpu.sync_copy(s_loc.at[pl.ds(0, LROWS * 128)],
                    s_out.at[pl.ds(lo * 128, LROWS * 128)])
    pltpu.sync_copy(c_loc.at[pl.ds(0, LROWS * 8)],
                    c_out.at[pl.ds(wid * LROWS * 8, LROWS * 8)])
    pltpu.sync_copy(q_loc.at[pl.ds(0, LROWS * 16)],
                    q_out.at[pl.ds(wid * LROWS * 16, LROWS * 16)])


@functools.lru_cache(maxsize=None)
def _get_hist_call():
  return functools.partial(
    pl.kernel,
    mesh=plsc.VectorSubcoreMesh(core_axis_name="c", subcore_axis_name="s"),
    compiler_params=pltpu.CompilerParams(needs_layout_passes=False),
    out_type=[
        jax.ShapeDtypeStruct((P * D,), jnp.float32),
        jax.ShapeDtypeStruct((P * 8,), jnp.float32),
        jax.ShapeDtypeStruct((P * 16,), jnp.float32),
    ],
    scratch_types=[
        pltpu.VMEM((704 * D,), jnp.float32),
        pltpu.VMEM((5648,), jnp.float32),
        pltpu.VMEM((11280,), jnp.float32),
        pltpu.VMEM((2 * CH,), jnp.int32),
        pltpu.VMEM((2 * (CH + GB),), jnp.int32),
        pltpu.VMEM((2 * (CH + GB),), jnp.int32),
        pltpu.VMEM((2, GB, D), jnp.float32),
        pltpu.SemaphoreType.DMA,
        pltpu.SemaphoreType.DMA,
        pltpu.SemaphoreType.DMA,
        pltpu.SemaphoreType.DMA,
    ],
  )(_hist_body)


def _stats_body(s_ref, wc_ref, wq_ref, mean_ref, tsum_ref, tsc_ref):
    S = s_ref[0]
    C = wc_ref[0][:, 0:1]
    Q = jnp.sum(wq_ref[0], axis=1, keepdims=True)
    mean = S / jnp.maximum(C, 1.0)
    mean_ref[...] = mean[None]
    tsum = jnp.sum(S, axis=0, keepdims=True)
    tsum_ref[...] = tsum[None]
    tcnt = jnp.sum(C)
    tQ = jnp.sum(Q)
    tmean = tsum / jnp.maximum(tcnt, 1.0)
    d2 = jnp.sum((mean - tmean) ** 2, axis=1, keepdims=True)
    msqn = jnp.sum(C * d2)
    m2 = jnp.sum(mean * mean, axis=1, keepdims=True)
    rsqn = tQ - jnp.sum(C * m2)
    io = lax.broadcasted_iota(jnp.int32, (1, 1, 8), 2)
    tsc_ref[...] = jnp.where(
        io == 0, tcnt,
        jnp.where(io == 1, tQ, jnp.where(io == 2, msqn, rsqn)))


_stats_call = pl.pallas_call(
    _stats_body,
    grid=(NG,),
    in_specs=[
        pl.BlockSpec((1, NL, D), lambda g: (g, 0, 0)),
        pl.BlockSpec((1, NL, 8), lambda g: (g, 0, 0)),
        pl.BlockSpec((1, NL, 16), lambda g: (g, 0, 0)),
    ],
    out_specs=[
        pl.BlockSpec((1, NL, D), lambda g: (g, 0, 0)),
        pl.BlockSpec((1, 1, D), lambda g: (g, 0, 0)),
        pl.BlockSpec((1, 1, 8), lambda g: (g, 0, 0)),
    ],
    out_shape=[
        jax.ShapeDtypeStruct((NG, NL, D), jnp.float32),
        jax.ShapeDtypeStruct((NG, 1, D), jnp.float32),
        jax.ShapeDtypeStruct((NG, 1, 8), jnp.float32),
    ],
)


def _alpha_body(tsum_ref, tsc_ref, ab_ref):
    tcnt = tsc_ref[:, 0:1]
    tQ = tsc_ref[:, 1:2]
    msqn = tsc_ref[:, 2:3]
    rsqn = tsc_ref[:, 3:4]
    gi = lax.broadcasted_iota(jnp.int32, (NG, 1), 0)
    tg = (gi // TPW) + 32 * (gi % TPW)
    is_tr = tg < SPLIT
    is_te = (tg >= SPLIT) & (tg < NT)
    te_cnt = jnp.sum(jnp.where(is_te, tcnt, 0.0))
    te_sum = jnp.sum(jnp.where(is_te, tsum_ref[...], 0.0), axis=0,
                     keepdims=True)
    te_Q = jnp.sum(jnp.where(is_te, tQ, 0.0))
    te_mean = te_sum / jnp.maximum(te_cnt, 1.0)
    test_var = (te_Q - te_cnt * jnp.sum(te_mean * te_mean)) / jnp.maximum(
        1.0, te_cnt - 1.0)
    den = jnp.maximum(1.0, tcnt - 1.0)
    msq = msqn / den
    rsq = rsqn / den
    a_sq = (test_var - msq) / jnp.maximum(1e-6, rsq)
    pos = a_sq > 0
    alpha = jnp.where(pos, jnp.sqrt(jnp.where(pos, a_sq, 1.0)), 0.0)
    A = jnp.where(is_tr, alpha, 1.0)
    B = jnp.where(is_tr, 1.0 - alpha, 0.0)
    io = lax.broadcasted_iota(jnp.int32, (NG, 8), 1)
    ab_ref[...] = jnp.where(io == 0, A, jnp.where(io == 1, B, 0.0))


_alpha_call = pl.pallas_call(
    _alpha_body,
    out_shape=jax.ShapeDtypeStruct((NG, 8), jnp.float32),
)


def _norm_body(x_hbm, keys_hbm, mean_hbm, a_hbm, b_hbm, out_hbm,
               av, bv, kb, mrows, xrows, orows, sem):
    wid = lax.axis_index("s") * 2 + lax.axis_index("c")
    pltpu.sync_copy(a_hbm, av)
    pltpu.sync_copy(b_hbm, bv)
    per = N // NW
    start = (wid * per // GB) * GB
    nxt = jnp.where(wid == NW - 1, N, ((wid + 1) * per // GB) * GB)
    nb = (nxt - start + GB - 1) // GB

    def cb(b, _):
        base = jnp.minimum(start + b * GB, N - GB)
        pltpu.sync_copy(keys_hbm.at[pl.ds(base, GB)], kb)
        cp = pltpu.async_copy(mean_hbm.at[kb], mrows, sem)
        pltpu.sync_copy(x_hbm.at[pl.ds(base, GB)], xrows)
        cp.wait()

        def ub(u, _):
            kv = kb[pl.ds(u * 16, 16)]
            tix = kv // NL
            aa = plsc.load_gather(av, [tix])
            bb = plsc.load_gather(bv, [tix])
            for r in range(16):
                af = jnp.full((16,), aa[r], jnp.float32)
                bf = jnp.full((16,), bb[r], jnp.float32)
                row = u * 16 + r
                for c in range(8):
                    sl = pl.ds(c * 16, 16)
                    orows[row, sl] = bf * mrows[row, sl] + af * xrows[row, sl]
            return 0

        lax.fori_loop(0, GB // 16, ub, 0)
        pltpu.sync_copy(orows, out_hbm.at[pl.ds(base, GB)])
        return 0

    lax.fori_loop(0, nb, cb, 0)


@functools.lru_cache(maxsize=None)
def _get_norm_call():
  return functools.partial(
    pl.kernel,
    mesh=plsc.VectorSubcoreMesh(core_axis_name="c", subcore_axis_name="s"),
    compiler_params=pltpu.CompilerParams(needs_layout_passes=False),
    out_type=jax.ShapeDtypeStruct((N, D), jnp.float32),
    scratch_types=[
        pltpu.VMEM((NG,), jnp.float32),
        pltpu.VMEM((NG,), jnp.float32),
        pltpu.VMEM((GB,), jnp.int32),
        pltpu.VMEM((GB, D), jnp.float32),
        pltpu.VMEM((GB, D), jnp.float32),
        pltpu.VMEM((GB, D), jnp.float32),
        pltpu.SemaphoreType.DMA,
    ],
  )(_norm_body)


def kernel(x, labels, times):
    t32 = times.astype(jnp.int32)
    l32 = labels.astype(jnp.int32)
    keys = _keys_call(t32.reshape(500, 1000),
                      l32.reshape(500, 1000)).reshape(N)
    Sf, Cf, Qf = _get_hist_call()(x, keys)
    S3 = Sf.reshape(NG, NL, D)
    C3 = Cf.reshape(NG, NL, 8)
    Q3 = Qf.reshape(NG, NL, 16)
    mean3, tsum3, tsc3 = _stats_call(S3, C3, Q3)
    ab = _alpha_call(tsum3.reshape(NG, D), tsc3.reshape(NG, 8))
    A = ab[:, 0]
    B = ab[:, 1]
    return _get_norm_call()(x, keys, mean3.reshape(P, D), A, B)
